# baseline (speedup 1.0000x reference)
import functools

import jax
import jax.numpy as jnp
from jax import lax
from jax.experimental import pallas as pl
from jax.experimental.pallas import tpu as pltpu
from jax.experimental.pallas import tpu_sc as plsc

_NC = 2
_NS = 16
_NW = _NC * _NS

_K = 80
_CH = 128
_ZR = 80


def _fill_f32(ref, rows, cols, val):
    groups = cols // 16

    def body(i, carry):
        r = i // groups
        k = i % groups
        ref[r, pl.ds(k * 16, 16)] = jnp.full((16,), val, jnp.float32)
        return carry

    lax.fori_loop(0, rows * groups, body, 0)


def _make_sc_agg(n, d):
    rows_per_tile = -(-n // (_NS * _ZR)) * _ZR
    npad = rows_per_tile * _NS
    nz = rows_per_tile // _ZR
    nblk = _CH // 8

    out_type = [
        jax.ShapeDtypeStruct((_NC, npad, d), jnp.float32),
    ]

    scratch = [
        pltpu.VMEM((2, 8, _K), jnp.int32),
        pltpu.VMEM((2, 8, _K), jnp.int32),
        pltpu.VMEM((4, _K, d), jnp.float32),
        pltpu.SemaphoreType.DMA,
        pltpu.SemaphoreType.DMA,
        pltpu.SemaphoreType.DMA,
        pltpu.SemaphoreType.DMA,
        pltpu.SemaphoreType.DMA,
        pltpu.SemaphoreType.DMA,
        pltpu.SemaphoreType.DMA,
        pltpu.SemaphoreType.DMA,
        pltpu.VMEM_SHARED((npad, d), jnp.float32),
    ]

    mesh = plsc.VectorSubcoreMesh(core_axis_name="c", subcore_axis_name="s")

    def body(h_hbm, src_hbm, dst_hbm, aggp_hbm,
             sidx, didx, rows_v, g0, g1, g2, g3, s0, s1, s2, s3, agg_sh):
        c = lax.axis_index("c")
        s = lax.axis_index("s")
        wid = s * _NC + c
        r0 = pl.multiple_of(s * rows_per_tile, _ZR)

        stg = rows_v.at[0, pl.ds(0, _ZR)]
        _fill_f32(stg, _ZR, d, 0.0)
        for i in range(nz):
            pltpu.sync_copy(stg, agg_sh.at[pl.ds(r0 + i * _ZR, _ZR)])

        plsc.subcore_barrier()

        gsems = (g0, g1, g2, g3)
        ssems = (s0, s1, s2, s3)
        descs = {}
        sdescs = {}

        def stage(b, slot):
            pltpu.sync_copy(src_hbm.at[wid, pl.ds(b * 8, 8)], sidx.at[slot])
            pltpu.sync_copy(dst_hbm.at[wid, pl.ds(b * 8, 8)], didx.at[slot])

        def fire(j):
            bslot = (j // 8) % 2
            descs[j] = pltpu.async_copy(h_hbm.at[sidx.at[bslot, j % 8]],
                                        rows_v.at[j % 4], gsems[j % 4])

        def drain(j):
            descs.pop(j).wait()

        def scatter(j):
            bslot = (j // 8) % 2
            sdescs[j] = pltpu.async_copy(rows_v.at[j % 4],
                                         agg_sh.at[didx.at[bslot, j % 8]],
                                         ssems[j % 4], add=True)

        def scat_wait(j):
            sdescs.pop(j).wait()

        total = nblk * 8
        stage(0, 0)
        fire(0)
        fire(1)
        for b in range(nblk):
            if b + 1 < nblk:
                stage(b + 1, (b + 1) % 2)
            for jj in range(8):
                j = b * 8 + jj
                drain(j)
                scatter(j)
                if j - 2 >= 0:
                    scat_wait(j - 2)
                if j + 2 < total:
                    fire(j + 2)
        scat_wait(total - 2)
        scat_wait(total - 1)

        plsc.subcore_barrier()

        for i in range(nz):
            rr = r0 + i * _ZR
            pltpu.sync_copy(agg_sh.at[pl.ds(rr, _ZR)], stg)
            pltpu.sync_copy(stg, aggp_hbm.at[c, pl.ds(rr, _ZR)])

    return pl.kernel(body, out_type=out_type, mesh=mesh,
                     scratch_types=scratch), npad


def _make_sc_deg(n, d):
    rows_per_tile = -(-n // (_NS * _ZR)) * _ZR
    npad = rows_per_tile * _NS
    nz = rows_per_tile // _ZR
    nblk = _CH // 8

    out_type = [jax.ShapeDtypeStruct((_NC, npad, d), jnp.float32)]

    scratch = [
        pltpu.VMEM((2, 8, _K), jnp.int32),
        pltpu.VMEM((_K, d), jnp.float32),
        pltpu.VMEM((_ZR, d), jnp.float32),
        pltpu.VMEM_SHARED((npad, d), jnp.float32),
    ]

    mesh = plsc.VectorSubcoreMesh(core_axis_name="c", subcore_axis_name="s")

    def body(dst_hbm, degp_hbm, didx, ones_v, degst_v, deg_sh):
        c = lax.axis_index("c")
        s = lax.axis_index("s")
        wid = s * _NC + c
        r0 = pl.multiple_of(s * rows_per_tile, _ZR)

        _fill_f32(ones_v, _K, d, 1.0)
        _fill_f32(degst_v, _ZR, d, 0.0)
        for i in range(nz):
            pltpu.sync_copy(degst_v, deg_sh.at[pl.ds(r0 + i * _ZR, _ZR)])
        plsc.subcore_barrier()

        def stage(b, slot):
            pltpu.sync_copy(dst_hbm.at[wid, pl.ds(b * 8, 8)], didx.at[slot])

        stage(0, 0)
        for b in range(nblk):
            if b + 1 < nblk:
                stage(b + 1, (b + 1) % 2)
            for jj in range(8):
                pltpu.sync_copy(ones_v, deg_sh.at[didx.at[b % 2, jj]],
                                add=True)

        plsc.subcore_barrier()
        for i in range(nz):
            rr = r0 + i * _ZR
            pltpu.sync_copy(deg_sh.at[pl.ds(rr, _ZR)], degst_v)
            pltpu.sync_copy(degst_v, degp_hbm.at[c, pl.ds(rr, _ZR)])

    return pl.kernel(body, out_type=out_type, mesh=mesh,
                     scratch_types=scratch)


def _tc_layer_body(n, npad, aggp_ref, degp_ref, w_ref, b_ref, g_ref, be_ref,
                   out_ref):
    agg = (aggp_ref[0] + aggp_ref[1])[0:n]
    deg = (degp_ref[0] + degp_ref[1])[0:n, 0:1]
    agg = agg / jnp.maximum(deg, 1.0)
    h = jnp.dot(agg, w_ref[...], preferred_element_type=jnp.float32)
    h = h + b_ref[...]
    mean = jnp.mean(h, axis=0, keepdims=True)
    var = jnp.mean((h - mean) ** 2, axis=0, keepdims=True)
    h = (h - mean) * lax.rsqrt(var + 1e-5) * g_ref[...] + be_ref[...]
    out_ref[...] = jnp.maximum(h, 0.0)


def _tc_final_body(n, npad, aggp_ref, degp_ref, w_ref, b_ref, g_ref, be_ref,
                   wc_ref, bc_ref, out_ref):
    agg = (aggp_ref[0] + aggp_ref[1])[0:n]
    deg = (degp_ref[0] + degp_ref[1])[0:n, 0:1]
    agg = agg / jnp.maximum(deg, 1.0)
    h = jnp.dot(agg, w_ref[...], preferred_element_type=jnp.float32)
    h = h + b_ref[...]
    mean = jnp.mean(h, axis=0, keepdims=True)
    var = jnp.mean((h - mean) ** 2, axis=0, keepdims=True)
    h = (h - mean) * lax.rsqrt(var + 1e-5) * g_ref[...] + be_ref[...]
    h = jnp.maximum(h, 0.0)
    logits = jnp.dot(h, wc_ref[...], preferred_element_type=jnp.float32)
    logits = logits + bc_ref[...]
    m = jnp.max(logits, axis=1, keepdims=True)
    z = logits - m
    lse = jnp.log(jnp.sum(jnp.exp(z), axis=1, keepdims=True))
    out_ref[...] = z - lse


def kernel(x, edge_index, W1, b1, gamma1, beta1, W2, b2, gamma2, beta2,
           Wc, bc):
    n, d = x.shape
    e = edge_index.shape[1]
    c_out = Wc.shape[1]

    sc_agg_deg, npad = _make_sc_agg(n, d)

    e_pad = _NW * _CH * _K
    pad = e_pad - e
    ar = jnp.arange(pad, dtype=jnp.int32)
    srcp = jnp.concatenate([edge_index[0], ar % n])
    dstp = jnp.concatenate([edge_index[1], n + ar % (npad - n)])
    src3 = srcp.reshape(_NW, _CH, _K)
    dst3 = dstp.reshape(_NW, _CH, _K)

    aggp1, = sc_agg_deg(x, src3, dst3)
    degp, = _make_sc_deg(n, d)(dst3)

    h1 = pl.pallas_call(
        functools.partial(_tc_layer_body, n, npad),
        out_shape=jax.ShapeDtypeStruct((n, d), jnp.float32),
    )(aggp1, degp, W1, b1.reshape(1, d), gamma1.reshape(1, d),
      beta1.reshape(1, d))

    aggp2, = sc_agg_deg(h1, src3, dst3)

    out = pl.pallas_call(
        functools.partial(_tc_final_body, n, npad),
        out_shape=jax.ShapeDtypeStruct((n, c_out), jnp.float32),
    )(aggp2, degp, W2, b2.reshape(1, d), gamma2.reshape(1, d),
      beta2.reshape(1, d), Wc, bc.reshape(1, c_out))

    return out


# speedup vs baseline: 1.0398x; 1.0398x over previous
import functools

import jax
import jax.numpy as jnp
from jax import lax
from jax.experimental import pallas as pl
from jax.experimental.pallas import tpu as pltpu
from jax.experimental.pallas import tpu_sc as plsc

_NC = 2
_NS = 16
_NW = _NC * _NS

_K = 125
_CH = 80
_ZR = 80


def _fill_f32(ref, rows, cols, val):
    groups = cols // 16

    def body(i, carry):
        r = i // groups
        k = i % groups
        ref[r, pl.ds(k * 16, 16)] = jnp.full((16,), val, jnp.float32)
        return carry

    lax.fori_loop(0, rows * groups, body, 0)


def _make_sc_agg(n, d):
    rows_per_tile = -(-n // (_NS * _ZR)) * _ZR
    npad = rows_per_tile * _NS
    nz = rows_per_tile // _ZR
    nblk = _CH // 8

    out_type = [
        jax.ShapeDtypeStruct((_NC, npad, d), jnp.float32),
    ]

    scratch = [
        pltpu.VMEM((2, 8, _K), jnp.int32),
        pltpu.VMEM((2, 8, _K), jnp.int32),
        pltpu.VMEM((2, _K, d), jnp.float32),
        pltpu.SemaphoreType.DMA,
        pltpu.SemaphoreType.DMA,
        pltpu.VMEM_SHARED((npad, d), jnp.float32),
    ]

    mesh = plsc.VectorSubcoreMesh(core_axis_name="c", subcore_axis_name="s")

    def body(h_hbm, src_hbm, dst_hbm, aggp_hbm,
             sidx, didx, rows_v, sem0, sem1, agg_sh):
        c = lax.axis_index("c")
        s = lax.axis_index("s")
        wid = s * _NC + c
        r0 = pl.multiple_of(s * rows_per_tile, _ZR)

        stg = rows_v.at[0, pl.ds(0, _ZR)]
        _fill_f32(stg, _ZR, d, 0.0)
        for i in range(nz):
            pltpu.sync_copy(stg, agg_sh.at[pl.ds(r0 + i * _ZR, _ZR)])

        plsc.subcore_barrier()

        sems = (sem0, sem1)
        descs = {}

        def stage(b, slot):
            pltpu.sync_copy(src_hbm.at[wid, pl.ds(b * 8, 8)], sidx.at[slot])
            pltpu.sync_copy(dst_hbm.at[wid, pl.ds(b * 8, 8)], didx.at[slot])

        def fire(j):
            bslot = (j // 8) % 2
            descs[j] = pltpu.async_copy(h_hbm.at[sidx.at[bslot, j % 8]],
                                        rows_v.at[j % 2], sems[j % 2])

        def drain(j):
            descs.pop(j).wait()

        def scatter(j):
            bslot = (j // 8) % 2
            pltpu.sync_copy(rows_v.at[j % 2], agg_sh.at[didx.at[bslot, j % 8]],
                            add=True)

        total = nblk * 8
        stage(0, 0)
        fire(0)
        for b in range(nblk):
            if b + 1 < nblk:
                stage(b + 1, (b + 1) % 2)
            for jj in range(8):
                j = b * 8 + jj
                if j + 1 < total:
                    fire(j + 1)
                drain(j)
                scatter(j)

        plsc.subcore_barrier()

        for i in range(nz):
            rr = r0 + i * _ZR
            pltpu.sync_copy(agg_sh.at[pl.ds(rr, _ZR)], stg)
            pltpu.sync_copy(stg, aggp_hbm.at[c, pl.ds(rr, _ZR)])

    return pl.kernel(body, out_type=out_type, mesh=mesh,
                     scratch_types=scratch), npad


def _make_sc_deg(n, d):
    rows_per_tile = -(-n // (_NS * _ZR)) * _ZR
    npad = rows_per_tile * _NS
    nz = rows_per_tile // _ZR
    nblk = _CH // 8

    out_type = [jax.ShapeDtypeStruct((_NC, npad, d), jnp.float32)]

    scratch = [
        pltpu.VMEM((2, 8, _K), jnp.int32),
        pltpu.VMEM((_K, d), jnp.float32),
        pltpu.VMEM((_ZR, d), jnp.float32),
        pltpu.VMEM_SHARED((npad, d), jnp.float32),
    ]

    mesh = plsc.VectorSubcoreMesh(core_axis_name="c", subcore_axis_name="s")

    def body(dst_hbm, degp_hbm, didx, ones_v, degst_v, deg_sh):
        c = lax.axis_index("c")
        s = lax.axis_index("s")
        wid = s * _NC + c
        r0 = pl.multiple_of(s * rows_per_tile, _ZR)

        _fill_f32(ones_v, _K, d, 1.0)
        _fill_f32(degst_v, _ZR, d, 0.0)
        for i in range(nz):
            pltpu.sync_copy(degst_v, deg_sh.at[pl.ds(r0 + i * _ZR, _ZR)])
        plsc.subcore_barrier()

        def stage(b, slot):
            pltpu.sync_copy(dst_hbm.at[wid, pl.ds(b * 8, 8)], didx.at[slot])

        stage(0, 0)
        for b in range(nblk):
            if b + 1 < nblk:
                stage(b + 1, (b + 1) % 2)
            for jj in range(8):
                pltpu.sync_copy(ones_v, deg_sh.at[didx.at[b % 2, jj]],
                                add=True)

        plsc.subcore_barrier()
        for i in range(nz):
            rr = r0 + i * _ZR
            pltpu.sync_copy(deg_sh.at[pl.ds(rr, _ZR)], degst_v)
            pltpu.sync_copy(degst_v, degp_hbm.at[c, pl.ds(rr, _ZR)])

    return pl.kernel(body, out_type=out_type, mesh=mesh,
                     scratch_types=scratch)


def _tc_layer_body(n, npad, aggp_ref, degp_ref, w_ref, b_ref, g_ref, be_ref,
                   out_ref):
    agg = (aggp_ref[0] + aggp_ref[1])[0:n]
    deg = (degp_ref[0] + degp_ref[1])[0:n, 0:1]
    agg = agg / jnp.maximum(deg, 1.0)
    h = jnp.dot(agg, w_ref[...], preferred_element_type=jnp.float32)
    h = h + b_ref[...]
    mean = jnp.mean(h, axis=0, keepdims=True)
    var = jnp.mean((h - mean) ** 2, axis=0, keepdims=True)
    h = (h - mean) * lax.rsqrt(var + 1e-5) * g_ref[...] + be_ref[...]
    out_ref[...] = jnp.maximum(h, 0.0)


def _tc_final_body(n, npad, aggp_ref, degp_ref, w_ref, b_ref, g_ref, be_ref,
                   wc_ref, bc_ref, out_ref):
    agg = (aggp_ref[0] + aggp_ref[1])[0:n]
    deg = (degp_ref[0] + degp_ref[1])[0:n, 0:1]
    agg = agg / jnp.maximum(deg, 1.0)
    h = jnp.dot(agg, w_ref[...], preferred_element_type=jnp.float32)
    h = h + b_ref[...]
    mean = jnp.mean(h, axis=0, keepdims=True)
    var = jnp.mean((h - mean) ** 2, axis=0, keepdims=True)
    h = (h - mean) * lax.rsqrt(var + 1e-5) * g_ref[...] + be_ref[...]
    h = jnp.maximum(h, 0.0)
    logits = jnp.dot(h, wc_ref[...], preferred_element_type=jnp.float32)
    logits = logits + bc_ref[...]
    m = jnp.max(logits, axis=1, keepdims=True)
    z = logits - m
    lse = jnp.log(jnp.sum(jnp.exp(z), axis=1, keepdims=True))
    out_ref[...] = z - lse


def kernel(x, edge_index, W1, b1, gamma1, beta1, W2, b2, gamma2, beta2,
           Wc, bc):
    n, d = x.shape
    e = edge_index.shape[1]
    c_out = Wc.shape[1]

    sc_agg_deg, npad = _make_sc_agg(n, d)

    e_pad = _NW * _CH * _K
    pad = e_pad - e
    ar = jnp.arange(pad, dtype=jnp.int32)
    srcp = jnp.concatenate([edge_index[0], ar % n])
    dstp = jnp.concatenate([edge_index[1], n + ar % (npad - n)])
    src3 = srcp.reshape(_NW, _CH, _K)
    dst3 = dstp.reshape(_NW, _CH, _K)

    aggp1, = sc_agg_deg(x, src3, dst3)
    degp, = _make_sc_deg(n, d)(dst3)

    h1 = pl.pallas_call(
        functools.partial(_tc_layer_body, n, npad),
        out_shape=jax.ShapeDtypeStruct((n, d), jnp.float32),
    )(aggp1, degp, W1, b1.reshape(1, d), gamma1.reshape(1, d),
      beta1.reshape(1, d))

    aggp2, = sc_agg_deg(h1, src3, dst3)

    out = pl.pallas_call(
        functools.partial(_tc_final_body, n, npad),
        out_shape=jax.ShapeDtypeStruct((n, c_out), jnp.float32),
    )(aggp2, degp, W2, b2.reshape(1, d), gamma2.reshape(1, d),
      beta2.reshape(1, d), Wc, bc.reshape(1, c_out))

    return out


# speedup vs baseline: 1.0509x; 1.0107x over previous
import functools

import jax
import jax.numpy as jnp
from jax import lax
from jax.experimental import pallas as pl
from jax.experimental.pallas import tpu as pltpu
from jax.experimental.pallas import tpu_sc as plsc

_NC = 2
_NS = 16
_NW = _NC * _NS

_K = 125
_CH = 80
_ZR = 80


def _fill_f32(ref, rows, cols, val):
    groups = cols // 16

    def body(i, carry):
        r = i // groups
        k = i % groups
        ref[r, pl.ds(k * 16, 16)] = jnp.full((16,), val, jnp.float32)
        return carry

    lax.fori_loop(0, rows * groups, body, 0)


def _make_sc_agg(n, d):
    rows_per_tile = -(-n // (_NS * _ZR)) * _ZR
    npad = rows_per_tile * _NS
    nz = rows_per_tile // _ZR
    nblk = _CH // 8

    out_type = [
        jax.ShapeDtypeStruct((_NC, npad, d), jnp.float32),
    ]

    scratch = [
        pltpu.VMEM((2, 8, _K), jnp.int32),
        pltpu.VMEM((2, 8, _K), jnp.int32),
        pltpu.VMEM((2, _K, d), jnp.float32),
        pltpu.SemaphoreType.DMA,
        pltpu.SemaphoreType.DMA,
        pltpu.VMEM_SHARED((npad, d), jnp.float32),
    ]

    mesh = plsc.VectorSubcoreMesh(core_axis_name="c", subcore_axis_name="s")

    def body(h_hbm, src_hbm, dst_hbm, aggp_hbm,
             sidx, didx, rows_v, sem0, sem1, agg_sh):
        c = lax.axis_index("c")
        s = lax.axis_index("s")
        wid = s * _NC + c
        r0 = pl.multiple_of(s * rows_per_tile, _ZR)

        stg = rows_v.at[0, pl.ds(0, _ZR)]
        _fill_f32(stg, _ZR, d, 0.0)
        for i in range(nz):
            pltpu.sync_copy(stg, agg_sh.at[pl.ds(r0 + i * _ZR, _ZR)])

        plsc.subcore_barrier()

        sems = (sem0, sem1)
        descs = {}

        def stage(b, slot):
            pltpu.sync_copy(src_hbm.at[wid, pl.ds(b * 8, 8)], sidx.at[slot])
            pltpu.sync_copy(dst_hbm.at[wid, pl.ds(b * 8, 8)], didx.at[slot])

        def fire(j):
            bslot = (j // 8) % 2
            descs[j] = pltpu.async_copy(h_hbm.at[sidx.at[bslot, j % 8]],
                                        rows_v.at[j % 2], sems[j % 2])

        def drain(j):
            descs.pop(j).wait()

        def scatter(j):
            bslot = (j // 8) % 2
            pltpu.sync_copy(rows_v.at[j % 2], agg_sh.at[didx.at[bslot, j % 8]],
                            add=True)

        total = nblk * 8
        stage(0, 0)
        fire(0)
        for b in range(nblk):
            if b + 1 < nblk:
                stage(b + 1, (b + 1) % 2)
            for jj in range(8):
                j = b * 8 + jj
                if j + 1 < total:
                    fire(j + 1)
                drain(j)
                scatter(j)

        plsc.subcore_barrier()

        for i in range(nz):
            rr = r0 + i * _ZR
            pltpu.sync_copy(agg_sh.at[pl.ds(rr, _ZR)], stg)
            pltpu.sync_copy(stg, aggp_hbm.at[c, pl.ds(rr, _ZR)])

    return pl.kernel(body, out_type=out_type, mesh=mesh,
                     scratch_types=scratch), npad


def _make_sc_deg(n, d):
    rows_per_tile = -(-n // (_NS * _ZR)) * _ZR
    npad = rows_per_tile * _NS
    nz = rows_per_tile // _ZR
    nblk = _CH // 8

    out_type = [jax.ShapeDtypeStruct((_NC, npad, d), jnp.float32)]

    scratch = [
        pltpu.VMEM((3, 8, _K), jnp.int32),
        pltpu.VMEM((_K, d), jnp.float32),
        pltpu.VMEM((_ZR, d), jnp.float32),
        pltpu.SemaphoreType.DMA,
        pltpu.SemaphoreType.DMA,
        pltpu.VMEM_SHARED((npad, d), jnp.float32),
    ]

    mesh = plsc.VectorSubcoreMesh(core_axis_name="c", subcore_axis_name="s")

    def body(dst_hbm, degp_hbm, didx, ones_v, degst_v, sem0, sem1, deg_sh):
        c = lax.axis_index("c")
        s = lax.axis_index("s")
        wid = s * _NC + c
        r0 = pl.multiple_of(s * rows_per_tile, _ZR)

        _fill_f32(ones_v, _K, d, 1.0)
        _fill_f32(degst_v, _ZR, d, 0.0)
        for i in range(nz):
            pltpu.sync_copy(degst_v, deg_sh.at[pl.ds(r0 + i * _ZR, _ZR)])
        plsc.subcore_barrier()

        sems = (sem0, sem1)
        sdescs = {}

        def stage(b, slot):
            pltpu.sync_copy(dst_hbm.at[wid, pl.ds(b * 8, 8)], didx.at[slot])

        total = nblk * 8
        stage(0, 0)
        for b in range(nblk):
            if b + 1 < nblk:
                stage(b + 1, (b + 1) % 3)
            for jj in range(8):
                j = b * 8 + jj
                sdescs[j] = pltpu.async_copy(
                    ones_v, deg_sh.at[didx.at[b % 3, jj]], sems[j % 2],
                    add=True)
                if j - 2 >= 0:
                    sdescs.pop(j - 2).wait()
        sdescs.pop(total - 2).wait()
        sdescs.pop(total - 1).wait()

        plsc.subcore_barrier()
        for i in range(nz):
            rr = r0 + i * _ZR
            pltpu.sync_copy(deg_sh.at[pl.ds(rr, _ZR)], degst_v)
            pltpu.sync_copy(degst_v, degp_hbm.at[c, pl.ds(rr, _ZR)])

    return pl.kernel(body, out_type=out_type, mesh=mesh,
                     scratch_types=scratch)


def _tc_layer_body(n, npad, aggp_ref, degp_ref, w_ref, b_ref, g_ref, be_ref,
                   out_ref):
    agg = (aggp_ref[0] + aggp_ref[1])[0:n]
    deg = (degp_ref[0] + degp_ref[1])[0:n, 0:1]
    agg = agg / jnp.maximum(deg, 1.0)
    h = jnp.dot(agg, w_ref[...], preferred_element_type=jnp.float32)
    h = h + b_ref[...]
    mean = jnp.mean(h, axis=0, keepdims=True)
    var = jnp.mean((h - mean) ** 2, axis=0, keepdims=True)
    h = (h - mean) * lax.rsqrt(var + 1e-5) * g_ref[...] + be_ref[...]
    out_ref[...] = jnp.maximum(h, 0.0)


def _tc_final_body(n, npad, aggp_ref, degp_ref, w_ref, b_ref, g_ref, be_ref,
                   wc_ref, bc_ref, out_ref):
    agg = (aggp_ref[0] + aggp_ref[1])[0:n]
    deg = (degp_ref[0] + degp_ref[1])[0:n, 0:1]
    agg = agg / jnp.maximum(deg, 1.0)
    h = jnp.dot(agg, w_ref[...], preferred_element_type=jnp.float32)
    h = h + b_ref[...]
    mean = jnp.mean(h, axis=0, keepdims=True)
    var = jnp.mean((h - mean) ** 2, axis=0, keepdims=True)
    h = (h - mean) * lax.rsqrt(var + 1e-5) * g_ref[...] + be_ref[...]
    h = jnp.maximum(h, 0.0)
    logits = jnp.dot(h, wc_ref[...], preferred_element_type=jnp.float32)
    logits = logits + bc_ref[...]
    m = jnp.max(logits, axis=1, keepdims=True)
    z = logits - m
    lse = jnp.log(jnp.sum(jnp.exp(z), axis=1, keepdims=True))
    out_ref[...] = z - lse


def kernel(x, edge_index, W1, b1, gamma1, beta1, W2, b2, gamma2, beta2,
           Wc, bc):
    n, d = x.shape
    e = edge_index.shape[1]
    c_out = Wc.shape[1]

    sc_agg_deg, npad = _make_sc_agg(n, d)

    e_pad = _NW * _CH * _K
    pad = e_pad - e
    ar = jnp.arange(pad, dtype=jnp.int32)
    srcp = jnp.concatenate([edge_index[0], ar % n])
    dstp = jnp.concatenate([edge_index[1], n + ar % (npad - n)])
    src3 = srcp.reshape(_NW, _CH, _K)
    dst3 = dstp.reshape(_NW, _CH, _K)

    aggp1, = sc_agg_deg(x, src3, dst3)
    degp, = _make_sc_deg(n, d)(dst3)

    h1 = pl.pallas_call(
        functools.partial(_tc_layer_body, n, npad),
        out_shape=jax.ShapeDtypeStruct((n, d), jnp.float32),
    )(aggp1, degp, W1, b1.reshape(1, d), gamma1.reshape(1, d),
      beta1.reshape(1, d))

    aggp2, = sc_agg_deg(h1, src3, dst3)

    out = pl.pallas_call(
        functools.partial(_tc_final_body, n, npad),
        out_shape=jax.ShapeDtypeStruct((n, c_out), jnp.float32),
    )(aggp2, degp, W2, b2.reshape(1, d), gamma2.reshape(1, d),
      beta2.reshape(1, d), Wc, bc.reshape(1, c_out))

    return out
